# baseline (speedup 1.0000x reference)
import functools

import jax
import jax.numpy as jnp
from jax import lax
from jax.experimental import pallas as pl
from jax.experimental.pallas import tpu as pltpu
from jax.experimental.pallas import tpu_sc as plsc

N = 10000
E = 160000
D = 256
HALF = 128
NQ = 4
QW = 64
NC = 2
NS = 16
N_CHUNK = 128
CHUNKS = 80
E_TILE = CHUNKS * N_CHUNK
E_PAD = E_TILE * NS
NBUF = 8
ROWS_TILE = 632
ACC_ROWS = N + 128


def _sc_aggregate_body(x4, srcs, dsts, zeros, out, src_v, dst_v,
                       buf0, buf1, buf2, buf3, buf4, buf5, buf6, buf7, hacc,
                       gs0, gs1, gs2, gs3, gs4, gs5, gs6, gs7,
                       ss0, ss1, ss2, ss3, ss4, ss5, ss6, ss7):
    c = lax.axis_index("c")
    s = lax.axis_index("s")
    bufs = (buf0, buf1, buf2, buf3, buf4, buf5, buf6, buf7)
    gsems = (gs0, gs1, gs2, gs3, gs4, gs5, gs6, gs7)
    ssems = (ss0, ss1, ss2, ss3, ss4, ss5, ss6, ss7)

    pltpu.sync_copy(dsts.at[s], dst_v)

    local = jnp.minimum(s * ROWS_TILE, N - ROWS_TILE)

    def load_and_prime(f):
        pltpu.sync_copy(srcs.at[f * NS + s], src_v)
        for b in range(NBUF):
            pltpu.async_copy(x4.at[src_v.at[b]], bufs[b], gsems[b])

    load_and_prime(c * 2)

    for q in range(2):
        pltpu.sync_copy(zeros.at[c * NS + s],
                        hacc.at[pl.ds(local, ROWS_TILE)])
        plsc.subcore_barrier()

        def group(i, _):
            for b in range(NBUF):
                g = NBUF * i + b
                pltpu.make_async_copy(x4.at[src_v.at[g]], bufs[b],
                                      gsems[b]).wait()
                pltpu.async_copy(bufs[b], hacc.at[dst_v.at[g]], ssems[b],
                                 add=True)
            for b in range(NBUF):
                g = NBUF * i + b

                @pl.when(g + NBUF < CHUNKS)
                def _(b=b, g=g):
                    pltpu.make_async_copy(bufs[b], hacc.at[dst_v.at[g]],
                                          ssems[b]).wait()
                    pltpu.async_copy(x4.at[src_v.at[g + NBUF]], bufs[b],
                                     gsems[b])
            return 0

        lax.fori_loop(0, CHUNKS // NBUF, group, 0)
        for b in range(NBUF):
            pltpu.make_async_copy(bufs[b], hacc.at[dst_v.at[CHUNKS - NBUF + b]],
                                  ssems[b]).wait()
        plsc.subcore_barrier()

        if q == 0:
            load_and_prime(c * 2 + 1)

        pltpu.sync_copy(hacc.at[pl.ds(local, ROWS_TILE)],
                        out.at[pl.ds(c * N + local, ROWS_TILE),
                               pl.ds(q * QW, QW)])
        plsc.subcore_barrier()


@functools.lru_cache(maxsize=None)
def _make_sc_aggregate():
    return functools.partial(
        pl.kernel,
        out_type=jax.ShapeDtypeStruct((NC * N, HALF), jnp.float32),
        mesh=plsc.VectorSubcoreMesh(core_axis_name="c", subcore_axis_name="s"),
        scratch_types=[
            pltpu.VMEM((CHUNKS, N_CHUNK), jnp.int32),
            pltpu.VMEM((CHUNKS, N_CHUNK), jnp.int32),
            *([pltpu.VMEM((N_CHUNK, QW), jnp.float32)] * NBUF),
            pltpu.VMEM_SHARED((ACC_ROWS, QW), jnp.float32),
            *([pltpu.SemaphoreType.DMA] * (2 * NBUF)),
        ],
        compiler_params=pltpu.CompilerParams(use_tc_tiling_on_sc=False),
    )(_sc_aggregate_body)


ROW_BLK = 2000


def _mlp_body_split(ha_ref, hb_ref, xa_ref, xb_ref, wa_ref, ba_ref, wb_ref,
                    bb_ref, out_ref):
    h = jnp.concatenate([ha_ref[...] + xa_ref[...],
                         hb_ref[...] + xb_ref[...]], axis=1)
    t = jnp.maximum(
        jnp.dot(h, wa_ref[...], preferred_element_type=jnp.float32)
        + ba_ref[...], 0.0)
    o = jnp.dot(t, wb_ref[...], preferred_element_type=jnp.float32) + bb_ref[...]
    out_ref[0] = o[:, :HALF]
    out_ref[1] = o[:, HALF:]


def _mlp_body_full(ha_ref, hb_ref, xa_ref, xb_ref, wa_ref, ba_ref, wb_ref,
                   bb_ref, out_ref):
    h = jnp.concatenate([ha_ref[...] + xa_ref[...],
                         hb_ref[...] + xb_ref[...]], axis=1)
    t = jnp.maximum(
        jnp.dot(h, wa_ref[...], preferred_element_type=jnp.float32)
        + ba_ref[...], 0.0)
    out_ref[...] = (
        jnp.dot(t, wb_ref[...], preferred_element_type=jnp.float32) + bb_ref[...])


def _mlp(h2col, x2col, wa, ba, wb, bb, split_out):
    nblk = N // ROW_BLK
    in_specs = [
        pl.BlockSpec((ROW_BLK, HALF), lambda i: (i, 0)),
        pl.BlockSpec((ROW_BLK, HALF), lambda i: (i + nblk, 0)),
        pl.BlockSpec((ROW_BLK, HALF), lambda i: (i, 0)),
        pl.BlockSpec((ROW_BLK, HALF), lambda i: (i + nblk, 0)),
        pl.BlockSpec((D, D), lambda i: (0, 0)),
        pl.BlockSpec((1, D), lambda i: (0, 0)),
        pl.BlockSpec((D, D), lambda i: (0, 0)),
        pl.BlockSpec((1, D), lambda i: (0, 0)),
    ]
    if split_out:
        out_shape = jax.ShapeDtypeStruct((NC, N, HALF), jnp.float32)
        out_specs = pl.BlockSpec((NC, ROW_BLK, HALF), lambda i: (0, i, 0))
        body = _mlp_body_split
    else:
        out_shape = jax.ShapeDtypeStruct((N, D), jnp.float32)
        out_specs = pl.BlockSpec((ROW_BLK, D), lambda i: (i, 0))
        body = _mlp_body_full
    return pl.pallas_call(
        body,
        grid=(nblk,),
        in_specs=in_specs,
        out_specs=out_specs,
        out_shape=out_shape,
    )(h2col, h2col, x2col, x2col, wa, ba.reshape(1, D), wb, bb.reshape(1, D))


def kernel(x, edge_index, W1a, b1a, W1b, b1b, W2a, b2a, W2b, b2b):
    src = edge_index[0].astype(jnp.int32)
    dst = edge_index[1].astype(jnp.int32)
    npad = E_PAD - E
    src_p = jnp.concatenate([src, jnp.zeros((npad,), jnp.int32)])
    dst_p = jnp.concatenate(
        [dst, N + (jnp.arange(npad, dtype=jnp.int32) % 128)])
    src_r = src_p.reshape(NS, CHUNKS, N_CHUNK)
    srcs = jnp.concatenate(
        [2 * src_r + off for off in (0, 1, 2 * N, 2 * N + 1)], axis=0)
    dsts = dst_p.reshape(NS, CHUNKS, N_CHUNK)

    x2 = x.reshape(N, NC, HALF).transpose(1, 0, 2).reshape(NC * N, HALF)

    zeros = jnp.zeros((NC * NS, ROWS_TILE, QW), jnp.float32)
    sc_aggregate = _make_sc_aggregate()
    a1 = sc_aggregate(x2.reshape(NQ * N, QW), srcs, dsts, zeros)
    y1 = _mlp(a1, x2, W1a, b1a, W1b, b1b, split_out=True)
    y1f = y1.reshape(NC * N, HALF)
    a2 = sc_aggregate(y1f.reshape(NQ * N, QW), srcs, dsts, zeros)
    y2 = _mlp(a2, y1f, W2a, b2a, W2b, b2b, split_out=False)
    return y2


# speedup vs baseline: 2.1660x; 2.1660x over previous
import functools

import jax
import jax.numpy as jnp
from jax import lax
from jax.experimental import pallas as pl
from jax.experimental.pallas import tpu as pltpu
from jax.experimental.pallas import tpu_sc as plsc

N = 10000
E = 160000
D = 256
NQ = 4
QW = D // NQ
NC = 2
NS = 16
E_TILE = E // NS
N_CHUNK = 125
CHUNKS = E_TILE // N_CHUNK
NBUF = 8
ROWS_TILE = 632


def _sc_aggregate_body(x4, srcs, dsts, out, src_v, dst_v,
                       buf0, buf1, buf2, buf3, buf4, buf5, buf6, buf7, hacc,
                       gs0, gs1, gs2, gs3, gs4, gs5, gs6, gs7,
                       ss0, ss1, ss2, ss3, ss4, ss5, ss6, ss7):
    c = lax.axis_index("c")
    s = lax.axis_index("s")
    bufs = (buf0, buf1, buf2, buf3, buf4, buf5, buf6, buf7)
    gsems = (gs0, gs1, gs2, gs3, gs4, gs5, gs6, gs7)
    ssems = (ss0, ss1, ss2, ss3, ss4, ss5, ss6, ss7)

    pltpu.sync_copy(dsts.at[s], dst_v)

    local = jnp.minimum(s * ROWS_TILE, N - ROWS_TILE)

    def load_and_prime(f):
        pltpu.sync_copy(srcs.at[f * NS + s], src_v)
        for b in range(NBUF):
            pltpu.async_copy(x4.at[src_v.at[b]], bufs[b], gsems[b])

    load_and_prime(c * 2)

    for q in range(2):
        f = c * 2 + q
        pltpu.sync_copy(x4.at[pl.ds(f * N + local, ROWS_TILE)],
                        hacc.at[pl.ds(local, ROWS_TILE)])
        plsc.subcore_barrier()

        def group(i, _):
            for b in range(NBUF):
                g = NBUF * i + b
                pltpu.make_async_copy(x4.at[src_v.at[g]], bufs[b],
                                      gsems[b]).wait()
                pltpu.async_copy(bufs[b], hacc.at[dst_v.at[g]], ssems[b],
                                 add=True)
            for b in range(NBUF):
                g = NBUF * i + b

                @pl.when(g + NBUF < CHUNKS)
                def _(b=b, g=g):
                    pltpu.make_async_copy(bufs[b], hacc.at[dst_v.at[g]],
                                          ssems[b]).wait()
                    pltpu.async_copy(x4.at[src_v.at[g + NBUF]], bufs[b],
                                     gsems[b])
            return 0

        lax.fori_loop(0, CHUNKS // NBUF, group, 0)
        for b in range(NBUF):
            pltpu.make_async_copy(bufs[b], hacc.at[dst_v.at[CHUNKS - NBUF + b]],
                                  ssems[b]).wait()
        plsc.subcore_barrier()

        if q == 0:
            load_and_prime(c * 2 + 1)

        pltpu.sync_copy(hacc.at[pl.ds(local, ROWS_TILE)],
                        out.at[pl.ds(c * N + local, ROWS_TILE),
                               pl.ds(q * QW, QW)])
        plsc.subcore_barrier()


@functools.lru_cache(maxsize=None)
def _make_sc_aggregate():
    return functools.partial(
        pl.kernel,
        out_type=jax.ShapeDtypeStruct((NC * N, 2 * QW), jnp.float32),
        mesh=plsc.VectorSubcoreMesh(core_axis_name="c", subcore_axis_name="s"),
        scratch_types=[
            pltpu.VMEM((CHUNKS, N_CHUNK), jnp.int32),
            pltpu.VMEM((CHUNKS, N_CHUNK), jnp.int32),
            *([pltpu.VMEM((N_CHUNK, QW), jnp.float32)] * NBUF),
            pltpu.VMEM_SHARED((N, QW), jnp.float32),
            *([pltpu.SemaphoreType.DMA] * (2 * NBUF)),
        ],
        compiler_params=pltpu.CompilerParams(use_tc_tiling_on_sc=False),
    )(_sc_aggregate_body)


ROW_BLK = 2000


def _mlp_body_split(ha_ref, hb_ref, wa_ref, ba_ref, wb_ref, bb_ref,
                    out_ref):
    h = jnp.concatenate([ha_ref[...], hb_ref[...]], axis=1)
    t = jnp.maximum(
        jnp.dot(h, wa_ref[...], preferred_element_type=jnp.float32)
        + ba_ref[...], 0.0)
    o = jnp.dot(t, wb_ref[...], preferred_element_type=jnp.float32) + bb_ref[...]
    for i in range(NQ):
        out_ref[i] = o[:, i * QW:(i + 1) * QW]


def _mlp_body_full(ha_ref, hb_ref, wa_ref, ba_ref, wb_ref, bb_ref,
                   out_ref):
    h = jnp.concatenate([ha_ref[...], hb_ref[...]], axis=1)
    t = jnp.maximum(
        jnp.dot(h, wa_ref[...], preferred_element_type=jnp.float32)
        + ba_ref[...], 0.0)
    out_ref[...] = (
        jnp.dot(t, wb_ref[...], preferred_element_type=jnp.float32) + bb_ref[...])


def _mlp(h2col, wa, ba, wb, bb, split_out):
    grid = (N // ROW_BLK,)
    nblk = N // ROW_BLK
    in_specs = [
        pl.BlockSpec((ROW_BLK, 2 * QW), lambda i: (i, 0)),
        pl.BlockSpec((ROW_BLK, 2 * QW), lambda i: (i + nblk, 0)),
        pl.BlockSpec((D, D), lambda i: (0, 0)),
        pl.BlockSpec((1, D), lambda i: (0, 0)),
        pl.BlockSpec((D, D), lambda i: (0, 0)),
        pl.BlockSpec((1, D), lambda i: (0, 0)),
    ]
    if split_out:
        out_shape = jax.ShapeDtypeStruct((NQ, N, QW), jnp.float32)
        out_specs = pl.BlockSpec((NQ, ROW_BLK, QW), lambda i: (0, i, 0))
        body = _mlp_body_split
    else:
        out_shape = jax.ShapeDtypeStruct((N, D), jnp.float32)
        out_specs = pl.BlockSpec((ROW_BLK, D), lambda i: (i, 0))
        body = _mlp_body_full
    return pl.pallas_call(
        body,
        grid=grid,
        in_specs=in_specs,
        out_specs=out_specs,
        out_shape=out_shape,
    )(h2col, h2col, wa, ba.reshape(1, D), wb, bb.reshape(1, D))


def kernel(x, edge_index, W1a, b1a, W1b, b1b, W2a, b2a, W2b, b2b):
    src = edge_index[0].astype(jnp.int32)
    dst = edge_index[1].astype(jnp.int32)
    src_r = src.reshape(NS, CHUNKS, N_CHUNK)
    srcs = jnp.concatenate([src_r + f * N for f in range(NQ)], axis=0)
    dsts = dst.reshape(NS, CHUNKS, N_CHUNK)

    x4 = x.reshape(N, NQ, QW).transpose(1, 0, 2).reshape(NQ * N, QW)

    sc_aggregate = _make_sc_aggregate()
    h1 = sc_aggregate(x4, srcs, dsts)
    y1 = _mlp(h1, W1a, b1a, W1b, b1b, split_out=True)
    h2 = sc_aggregate(y1.reshape(NQ * N, QW), srcs, dsts)
    y2 = _mlp(h2, W2a, b2a, W2b, b2b, split_out=False)
    return y2


# speedup vs baseline: 2.2283x; 1.0288x over previous
import functools

import jax
import jax.numpy as jnp
from jax import lax
from jax.experimental import pallas as pl
from jax.experimental.pallas import tpu as pltpu
from jax.experimental.pallas import tpu_sc as plsc

N = 10000
E = 160000
D = 256
NQ = 4
QW = D // NQ
NC = 2
NS = 16
E_TILE = E // NS
N_CHUNK = 125
CHUNKS = E_TILE // N_CHUNK
NBUF = 8
ROWS_TILE = 632


def _sc_aggregate_body(x4, srcs, dsts, out, src_v, dst_v,
                       buf0, buf1, buf2, buf3, buf4, buf5, buf6, buf7, hacc,
                       gs0, gs1, gs2, gs3, gs4, gs5, gs6, gs7,
                       ss0, ss1, ss2, ss3, ss4, ss5, ss6, ss7):
    c = lax.axis_index("c")
    s = lax.axis_index("s")
    bufs = (buf0, buf1, buf2, buf3, buf4, buf5, buf6, buf7)
    gsems = (gs0, gs1, gs2, gs3, gs4, gs5, gs6, gs7)
    ssems = (ss0, ss1, ss2, ss3, ss4, ss5, ss6, ss7)

    pltpu.sync_copy(dsts.at[s], dst_v)

    local = jnp.minimum(s * ROWS_TILE, N - ROWS_TILE)

    def load_and_prime(f):
        pltpu.sync_copy(srcs.at[f * NS + s], src_v)
        for b in range(NBUF):
            pltpu.async_copy(x4.at[src_v.at[b]], bufs[b], gsems[b])

    load_and_prime(c * 2)

    for q in range(2):
        f = c * 2 + q
        pltpu.sync_copy(x4.at[pl.ds(f * N + local, ROWS_TILE)],
                        hacc.at[pl.ds(local, ROWS_TILE)])
        plsc.subcore_barrier()

        def group(i, _):
            for b in range(NBUF):
                g = NBUF * i + b
                pltpu.make_async_copy(x4.at[src_v.at[g]], bufs[b],
                                      gsems[b]).wait()
                pltpu.async_copy(bufs[b], hacc.at[dst_v.at[g]], ssems[b],
                                 add=True)
            for b in range(NBUF):
                g = NBUF * i + b

                @pl.when(g + NBUF < CHUNKS)
                def _(b=b, g=g):
                    pltpu.make_async_copy(bufs[b], hacc.at[dst_v.at[g]],
                                          ssems[b]).wait()
                    pltpu.async_copy(x4.at[src_v.at[g + NBUF]], bufs[b],
                                     gsems[b])
            return 0

        lax.fori_loop(0, CHUNKS // NBUF, group, 0)
        for b in range(NBUF):
            pltpu.make_async_copy(bufs[b], hacc.at[dst_v.at[CHUNKS - NBUF + b]],
                                  ssems[b]).wait()
        plsc.subcore_barrier()

        if q == 0:
            load_and_prime(c * 2 + 1)

        pltpu.sync_copy(hacc.at[pl.ds(local, ROWS_TILE)],
                        out.at[pl.ds(c * N + local, ROWS_TILE),
                               pl.ds(q * QW, QW)])
        plsc.subcore_barrier()


@functools.lru_cache(maxsize=None)
def _make_sc_aggregate():
    return functools.partial(
        pl.kernel,
        out_type=jax.ShapeDtypeStruct((NC * N, 2 * QW), jnp.float32),
        mesh=plsc.VectorSubcoreMesh(core_axis_name="c", subcore_axis_name="s"),
        scratch_types=[
            pltpu.VMEM((CHUNKS, N_CHUNK), jnp.int32),
            pltpu.VMEM((CHUNKS, N_CHUNK), jnp.int32),
            *([pltpu.VMEM((N_CHUNK, QW), jnp.float32)] * NBUF),
            pltpu.VMEM_SHARED((N, QW), jnp.float32),
            *([pltpu.SemaphoreType.DMA] * (2 * NBUF)),
        ],
        compiler_params=pltpu.CompilerParams(use_tc_tiling_on_sc=False),
    )(_sc_aggregate_body)


ROW_BLK = 2000


def _mlp_body_split(ha_ref, hb_ref, wa_ref, ba_ref, wb_ref, bb_ref,
                    out_ref):
    h = jnp.concatenate([ha_ref[...], hb_ref[...]], axis=1)
    t = jnp.maximum(
        jnp.dot(h, wa_ref[...], preferred_element_type=jnp.float32)
        + ba_ref[...], 0.0)
    o = jnp.dot(t, wb_ref[...], preferred_element_type=jnp.float32) + bb_ref[...]
    o4 = o.reshape(ROW_BLK // 2, 2, NQ, QW)
    for i in range(NQ):
        out_ref[i] = o4[:, :, i, :].reshape(ROW_BLK // 2, 2 * QW)


def _mlp_body_full(ha_ref, hb_ref, wa_ref, ba_ref, wb_ref, bb_ref,
                   out_ref):
    h = jnp.concatenate([ha_ref[...], hb_ref[...]], axis=1)
    t = jnp.maximum(
        jnp.dot(h, wa_ref[...], preferred_element_type=jnp.float32)
        + ba_ref[...], 0.0)
    out_ref[...] = (
        jnp.dot(t, wb_ref[...], preferred_element_type=jnp.float32) + bb_ref[...])


def _mlp(h2col, wa, ba, wb, bb, split_out):
    grid = (N // ROW_BLK,)
    nblk = N // ROW_BLK
    in_specs = [
        pl.BlockSpec((ROW_BLK, 2 * QW), lambda i: (i, 0)),
        pl.BlockSpec((ROW_BLK, 2 * QW), lambda i: (i + nblk, 0)),
        pl.BlockSpec((D, D), lambda i: (0, 0)),
        pl.BlockSpec((1, D), lambda i: (0, 0)),
        pl.BlockSpec((D, D), lambda i: (0, 0)),
        pl.BlockSpec((1, D), lambda i: (0, 0)),
    ]
    if split_out:
        out_shape = jax.ShapeDtypeStruct((NQ, N // 2, 2 * QW), jnp.float32)
        out_specs = pl.BlockSpec((NQ, ROW_BLK // 2, 2 * QW),
                                 lambda i: (0, i, 0))
        body = _mlp_body_split
    else:
        out_shape = jax.ShapeDtypeStruct((N, D), jnp.float32)
        out_specs = pl.BlockSpec((ROW_BLK, D), lambda i: (i, 0))
        body = _mlp_body_full
    return pl.pallas_call(
        body,
        grid=grid,
        in_specs=in_specs,
        out_specs=out_specs,
        out_shape=out_shape,
    )(h2col, h2col, wa, ba.reshape(1, D), wb, bb.reshape(1, D))


def kernel(x, edge_index, W1a, b1a, W1b, b1b, W2a, b2a, W2b, b2b):
    src = edge_index[0].astype(jnp.int32)
    dst = edge_index[1].astype(jnp.int32)
    src_r = src.reshape(NS, CHUNKS, N_CHUNK)
    srcs = jnp.concatenate([src_r + f * N for f in range(NQ)], axis=0)
    dsts = dst.reshape(NS, CHUNKS, N_CHUNK)

    x4 = x.reshape(N, NQ, QW).transpose(1, 0, 2).reshape(NQ * N, QW)

    sc_aggregate = _make_sc_aggregate()
    h1 = sc_aggregate(x4, srcs, dsts)
    y1 = _mlp(h1, W1a, b1a, W1b, b1b, split_out=True)
    h2 = sc_aggregate(y1.reshape(NQ * N, QW), srcs, dsts)
    y2 = _mlp(h2, W2a, b2a, W2b, b2b, split_out=False)
    return y2
